# baseline (speedup 1.0000x reference)
import functools

import jax
import jax.numpy as jnp
from jax import lax
from jax.experimental import pallas as pl
from jax.experimental.pallas import tpu as pltpu
from jax.experimental.pallas import tpu_sc as plsc

E = 64
K = 2
NUM_CORES = 2
NUM_SUBCORES = 16
NUM_WORKERS = NUM_CORES * NUM_SUBCORES
LANES = 16
ROW_BLK = 2048
GROUPS = 4


def _logits_body(x_ref, w_ref, b_ref, out_ref, aux_ref):
    acc = lax.dot_general(
        w_ref[...], x_ref[...],
        dimension_numbers=(((1,), (1,)), ((), ())),
        preferred_element_type=jnp.float32,
    ) + b_ref[...]
    out_ref[...] = acc
    m = jnp.max(acc, axis=0, keepdims=True)
    ssum = jnp.sum(jnp.exp(acc - m), axis=0, keepdims=True)
    aux_ref[...] = jnp.concatenate([m, ssum], axis=0)


def _tc_logits_t(xf, W, bc, row0, n_chunk):
    d = xf.shape[1]
    grid = n_chunk // ROW_BLK
    blk0 = row0 // ROW_BLK
    return pl.pallas_call(
        _logits_body,
        grid=(grid,),
        in_specs=[
            pl.BlockSpec((ROW_BLK, d), lambda i: (blk0 + i, 0)),
            pl.BlockSpec((E, d), lambda i: (0, 0)),
            pl.BlockSpec((E, 1), lambda i: (0, 0)),
        ],
        out_specs=[
            pl.BlockSpec((E, ROW_BLK), lambda i: (0, i)),
            pl.BlockSpec((2, ROW_BLK), lambda i: (0, i)),
        ],
        out_shape=[
            jax.ShapeDtypeStruct((E, n_chunk), jnp.float32),
            jax.ShapeDtypeStruct((2, n_chunk), jnp.float32),
        ],
    )(xf, W, bc)


def _router_body(rows_w, logits_hbm, aux_hbm, i1_hbm, i2_hbm, w1_hbm, w2_hbm,
                 buf, auxb, i1v, i2v, w1v, w2v):
    wid = lax.axis_index("s") * NUM_CORES + lax.axis_index("c")
    base = wid * rows_w
    pltpu.sync_copy(logits_hbm.at[:, pl.ds(base, rows_w)], buf)
    pltpu.sync_copy(aux_hbm.at[:, pl.ds(base, rows_w)], auxb)

    neg = jnp.float32(-3.4e38)

    def block(gb, carry):
        row0 = gb * (LANES * GROUPS)
        m1 = [jnp.full((LANES,), neg, jnp.float32) for _ in range(GROUPS)]
        m2 = [jnp.full((LANES,), neg, jnp.float32) for _ in range(GROUPS)]
        i1 = [jnp.zeros((LANES,), jnp.int32) for _ in range(GROUPS)]
        i2 = [jnp.zeros((LANES,), jnp.int32) for _ in range(GROUPS)]
        for e in range(E):
            col = jnp.full((LANES,), e, jnp.int32)
            for g in range(GROUPS):
                v = buf[e, pl.ds(row0 + g * LANES, LANES)]
                is1 = v > m1[g]
                is2 = v > m2[g]
                m2[g], i2[g] = (
                    jnp.where(is1, m1[g], jnp.where(is2, v, m2[g])),
                    jnp.where(is1, i1[g], jnp.where(is2, col, i2[g])),
                )
                m1[g] = jnp.where(is1, v, m1[g])
                i1[g] = jnp.where(is1, col, i1[g])
        for g in range(GROUPS):
            r0 = row0 + g * LANES
            mrow = auxb[0, pl.ds(r0, LANES)]
            inv = 1.0 / auxb[1, pl.ds(r0, LANES)]
            i1v[pl.ds(r0, LANES)] = i1[g]
            i2v[pl.ds(r0, LANES)] = i2[g]
            w1v[pl.ds(r0, LANES)] = jnp.exp(m1[g] - mrow) * inv
            w2v[pl.ds(r0, LANES)] = jnp.exp(m2[g] - mrow) * inv
        return carry

    lax.fori_loop(0, rows_w // (LANES * GROUPS), block, 0)

    pltpu.sync_copy(i1v, i1_hbm.at[pl.ds(base, rows_w)])
    pltpu.sync_copy(i2v, i2_hbm.at[pl.ds(base, rows_w)])
    pltpu.sync_copy(w1v, w1_hbm.at[pl.ds(base, rows_w)])
    pltpu.sync_copy(w2v, w2_hbm.at[pl.ds(base, rows_w)])


def _sc_router(logits_t, aux):
    n = logits_t.shape[1]
    rows_w = n // NUM_WORKERS
    mesh = plsc.VectorSubcoreMesh(
        core_axis_name="c", subcore_axis_name="s",
        num_cores=NUM_CORES, num_subcores=NUM_SUBCORES)
    return pl.kernel(
        functools.partial(_router_body, rows_w),
        out_type=(
            jax.ShapeDtypeStruct((n,), jnp.int32),
            jax.ShapeDtypeStruct((n,), jnp.int32),
            jax.ShapeDtypeStruct((n,), jnp.float32),
            jax.ShapeDtypeStruct((n,), jnp.float32),
        ),
        mesh=mesh,
        compiler_params=pltpu.CompilerParams(needs_layout_passes=False),
        scratch_types=[
            pltpu.VMEM((E, rows_w), jnp.float32),
            pltpu.VMEM((2, rows_w), jnp.float32),
            pltpu.VMEM((rows_w,), jnp.int32),
            pltpu.VMEM((rows_w,), jnp.int32),
            pltpu.VMEM((rows_w,), jnp.float32),
            pltpu.VMEM((rows_w,), jnp.float32),
        ],
    )(logits_t, aux)


CHUNK_SIZES = (8192, 8192, 8192, 8192)


def kernel(x, W, b):
    bsz, seq, d = x.shape
    n = bsz * seq
    xf = x.reshape(n, d)
    bc = b.reshape(E, 1)
    parts = []
    row0 = 0
    for n_chunk in CHUNK_SIZES:
        logits_t, aux = _tc_logits_t(xf, W, bc, row0, n_chunk)
        parts.append(_sc_router(logits_t, aux))
        row0 += n_chunk
    i1, i2, w1, w2 = (jnp.concatenate([p[j] for p in parts])
                      for j in range(4))
    idx = jnp.stack([i1, i2], axis=-1).reshape(bsz, seq, K)
    wts = jnp.stack([w1, w2], axis=-1).reshape(bsz, seq, K)
    return idx, wts


# speedup vs baseline: 1.0516x; 1.0516x over previous
import functools

import jax
import jax.numpy as jnp
from jax import lax
from jax.experimental import pallas as pl
from jax.experimental.pallas import tpu as pltpu
from jax.experimental.pallas import tpu_sc as plsc

E = 64
K = 2
NUM_CORES = 2
NUM_SUBCORES = 16
NUM_WORKERS = NUM_CORES * NUM_SUBCORES
LANES = 16
ROW_BLK = 2048
GROUPS = 4


def _logits_body(x_ref, w_ref, b_ref, out_ref, aux_ref):
    acc = lax.dot_general(
        w_ref[...], x_ref[...],
        dimension_numbers=(((1,), (1,)), ((), ())),
        preferred_element_type=jnp.float32,
    ) + b_ref[...]
    out_ref[...] = acc
    m = jnp.max(acc, axis=0, keepdims=True)
    ssum = jnp.sum(jnp.exp(acc - m), axis=0, keepdims=True)
    aux_ref[...] = jnp.concatenate([m, ssum], axis=0)


def _tc_logits_t(xf, W, bc, row0, n_chunk):
    d = xf.shape[1]
    grid = n_chunk // ROW_BLK
    blk0 = row0 // ROW_BLK
    return pl.pallas_call(
        _logits_body,
        grid=(grid,),
        in_specs=[
            pl.BlockSpec((ROW_BLK, d), lambda i: (blk0 + i, 0)),
            pl.BlockSpec((E, d), lambda i: (0, 0)),
            pl.BlockSpec((E, 1), lambda i: (0, 0)),
        ],
        out_specs=[
            pl.BlockSpec((E, ROW_BLK), lambda i: (0, i)),
            pl.BlockSpec((2, ROW_BLK), lambda i: (0, i)),
        ],
        out_shape=[
            jax.ShapeDtypeStruct((E, n_chunk), jnp.float32),
            jax.ShapeDtypeStruct((2, n_chunk), jnp.float32),
        ],
    )(xf, W, bc)


def _router_body(rows_w, logits_hbm, aux_hbm, i1_hbm, i2_hbm, w1_hbm, w2_hbm,
                 buf, auxb, i1v, i2v, w1v, w2v):
    wid = lax.axis_index("s") * NUM_CORES + lax.axis_index("c")
    base = wid * rows_w
    pltpu.sync_copy(logits_hbm.at[:, pl.ds(base, rows_w)], buf)
    pltpu.sync_copy(aux_hbm.at[:, pl.ds(base, rows_w)], auxb)

    neg = jnp.float32(-3.4e38)

    def block(gb, carry):
        row0 = gb * (LANES * GROUPS)
        m1 = [jnp.full((LANES,), neg, jnp.float32) for _ in range(GROUPS)]
        m2 = [jnp.full((LANES,), neg, jnp.float32) for _ in range(GROUPS)]
        i1 = [jnp.zeros((LANES,), jnp.int32) for _ in range(GROUPS)]
        i2 = [jnp.zeros((LANES,), jnp.int32) for _ in range(GROUPS)]
        for e in range(E):
            col = jnp.full((LANES,), e, jnp.int32)
            for g in range(GROUPS):
                v = buf[e, pl.ds(row0 + g * LANES, LANES)]
                is1 = v > m1[g]
                is2 = v > m2[g]
                m2[g], i2[g] = (
                    jnp.where(is1, m1[g], jnp.where(is2, v, m2[g])),
                    jnp.where(is1, i1[g], jnp.where(is2, col, i2[g])),
                )
                m1[g] = jnp.where(is1, v, m1[g])
                i1[g] = jnp.where(is1, col, i1[g])
        for g in range(GROUPS):
            r0 = row0 + g * LANES
            mrow = auxb[0, pl.ds(r0, LANES)]
            inv = 1.0 / auxb[1, pl.ds(r0, LANES)]
            i1v[pl.ds(r0, LANES)] = i1[g]
            i2v[pl.ds(r0, LANES)] = i2[g]
            w1v[pl.ds(r0, LANES)] = jnp.exp(m1[g] - mrow) * inv
            w2v[pl.ds(r0, LANES)] = jnp.exp(m2[g] - mrow) * inv
        return carry

    lax.fori_loop(0, rows_w // (LANES * GROUPS), block, 0)

    pltpu.sync_copy(i1v, i1_hbm.at[pl.ds(base, rows_w)])
    pltpu.sync_copy(i2v, i2_hbm.at[pl.ds(base, rows_w)])
    pltpu.sync_copy(w1v, w1_hbm.at[pl.ds(base, rows_w)])
    pltpu.sync_copy(w2v, w2_hbm.at[pl.ds(base, rows_w)])


def _sc_router(logits_t, aux):
    n = logits_t.shape[1]
    rows_w = n // NUM_WORKERS
    mesh = plsc.VectorSubcoreMesh(
        core_axis_name="c", subcore_axis_name="s",
        num_cores=NUM_CORES, num_subcores=NUM_SUBCORES)
    return pl.kernel(
        functools.partial(_router_body, rows_w),
        out_type=(
            jax.ShapeDtypeStruct((n,), jnp.int32),
            jax.ShapeDtypeStruct((n,), jnp.int32),
            jax.ShapeDtypeStruct((n,), jnp.float32),
            jax.ShapeDtypeStruct((n,), jnp.float32),
        ),
        mesh=mesh,
        compiler_params=pltpu.CompilerParams(needs_layout_passes=False),
        scratch_types=[
            pltpu.VMEM((E, rows_w), jnp.float32),
            pltpu.VMEM((2, rows_w), jnp.float32),
            pltpu.VMEM((rows_w,), jnp.int32),
            pltpu.VMEM((rows_w,), jnp.int32),
            pltpu.VMEM((rows_w,), jnp.float32),
            pltpu.VMEM((rows_w,), jnp.float32),
        ],
    )(logits_t, aux)


CHUNK_SIZES = (16384, 16384)


def kernel(x, W, b):
    bsz, seq, d = x.shape
    n = bsz * seq
    xf = x.reshape(n, d)
    bc = b.reshape(E, 1)
    parts = []
    row0 = 0
    for n_chunk in CHUNK_SIZES:
        logits_t, aux = _tc_logits_t(xf, W, bc, row0, n_chunk)
        parts.append(_sc_router(logits_t, aux))
        row0 += n_chunk
    i1, i2, w1, w2 = (jnp.concatenate([p[j] for p in parts])
                      for j in range(4))
    idx = jnp.stack([i1, i2], axis=-1).reshape(bsz, seq, K)
    wts = jnp.stack([w1, w2], axis=-1).reshape(bsz, seq, K)
    return idx, wts
